# Initial kernel scaffold; baseline (speedup 1.0000x reference)
import jax
import jax.numpy as jnp
from jax.experimental import pallas as pl


def kernel(dr, Z_i, Z_j, embeddings):
    raise NotImplementedError("write your pallas kernel here")


# speedup vs baseline: 12.1441x; 12.1441x over previous
import functools
import math

import jax
import jax.numpy as jnp
import numpy as np
from jax import lax
from jax.experimental import pallas as pl
from jax.experimental.pallas import tpu as pltpu
from jax.experimental.pallas import tpu_sc as plsc

N_NBRS = 1600000
N_SPECIES = 119
N_RADIAL = 5
N_BASIS = 7
R_MIN = 0.5
R_MAX = 6.0

ROW_PAD = 48

NC = 2
NS = 16
NW = NC * NS
PER_W = N_NBRS // NW
CHUNK = 80
NG = CHUNK // 16
ITERS = PER_W // CHUNK

BETTA = N_BASIS ** 2 / R_MAX ** 2
RAD_NORM = (2.0 * BETTA / math.pi) ** 0.25
EMBED_NORM = 1.0 / math.sqrt(N_BASIS)
LN_A = math.log(RAD_NORM * EMBED_NORM)
SHIFTS = [R_MIN + (R_MAX - R_MIN) / N_BASIS * k for k in range(N_BASIS)]

CUT_K = (math.pi / 12.0) ** 2
CUT_POLY = (2.3153174155601487e-05, -0.001385366693302603,
            0.041663578930696436, -0.4999990506281048, 0.9999999532476085)


def _radial_sc_body(dr_hbm, zi_hbm, zj_hbm, tab_hbm, out_hbm,
                    dr_v, zi_v, zj_v, idx_v, rows_v, out_v, sem):
    cid = lax.axis_index("c")
    sid = lax.axis_index("s")
    wid = sid * NC + cid
    base = wid * PER_W
    lane = lax.iota(jnp.int32, 16)

    def chunk(j, carry):
        cb = pl.multiple_of(base + j * CHUNK, 8)
        pltpu.sync_copy(dr_hbm.at[pl.ds(cb, CHUNK)], dr_v)
        pltpu.sync_copy(zi_hbm.at[pl.ds(cb, CHUNK)], zi_v)
        pltpu.sync_copy(zj_hbm.at[pl.ds(cb, CHUNK)], zj_v)
        for g in range(NG):
            sl = pl.ds(g * 16, 16)
            idx_v[sl] = zj_v[sl] * N_SPECIES + zi_v[sl]
        pltpu.async_copy(tab_hbm.at[idx_v], rows_v, sem).wait()
        for g in range(NG):
            sl = pl.ds(g * 16, 16)
            drv = dr_v[sl]
            rows = lane + g * 16
            basis = []
            for k in range(N_BASIS):
                d = drv - SHIFTS[k]
                basis.append(jnp.exp((LN_A - BETTA * d * d).astype(jnp.float32)))
            t = drv * drv * CUT_K
            c = jnp.float32(CUT_POLY[0])
            for coef in CUT_POLY[1:]:
                c = c * t + jnp.float32(coef)
            cut = c * c
            for r in range(N_RADIAL):
                acc = plsc.load_gather(
                    rows_v, [rows, jnp.full((16,), r * N_BASIS, jnp.int32)]
                ) * basis[0]
                for k in range(1, N_BASIS):
                    acc = acc + plsc.load_gather(
                        rows_v,
                        [rows, jnp.full((16,), r * N_BASIS + k, jnp.int32)],
                    ) * basis[k]
                plsc.store_scatter(
                    out_v, [rows, jnp.full((16,), r, jnp.int32)], acc * cut)
        pltpu.sync_copy(out_v, out_hbm.at[pl.ds(cb, CHUNK)])
        return carry

    lax.fori_loop(0, ITERS, chunk, 0)


_radial_sc = pl.kernel(
    _radial_sc_body,
    out_type=jax.ShapeDtypeStruct((N_NBRS, N_RADIAL), jnp.float32),
    mesh=plsc.VectorSubcoreMesh(core_axis_name="c", subcore_axis_name="s",
                                num_cores=NC, num_subcores=NS),
    scratch_types=[
        pltpu.VMEM((CHUNK,), jnp.float32),
        pltpu.VMEM((CHUNK,), jnp.int32),
        pltpu.VMEM((CHUNK,), jnp.int32),
        pltpu.VMEM((CHUNK,), jnp.int32),
        pltpu.VMEM((CHUNK, ROW_PAD), jnp.float32),
        pltpu.VMEM((CHUNK, N_RADIAL), jnp.float32),
        pltpu.SemaphoreType.DMA,
    ],
    compiler_params=pltpu.CompilerParams(use_tc_tiling_on_sc=False,
                                         needs_layout_passes=False),
)


def kernel(dr, Z_i, Z_j, embeddings):
    table = embeddings.reshape(N_SPECIES * N_SPECIES, N_RADIAL * N_BASIS)
    table = jnp.pad(table, ((0, 0), (0, ROW_PAD - N_RADIAL * N_BASIS)))
    return _radial_sc(dr.astype(jnp.float32), Z_i, Z_j, table)


# speedup vs baseline: 17.7575x; 1.4622x over previous
import math

import jax
import jax.numpy as jnp
from jax import lax
from jax.experimental import pallas as pl
from jax.experimental.pallas import tpu as pltpu
from jax.experimental.pallas import tpu_sc as plsc

N_NBRS = 1600000
N_SPECIES = 119
N_RADIAL = 5
N_BASIS = 7
R_MIN = 0.5
R_MAX = 6.0

ROW_PAD = 48

NC = 2
NS = 16
NW = NC * NS
PER_W = N_NBRS // NW
SUB = 80
NSUB = 5
CHUNK = SUB * NSUB
NG = CHUNK // 16
ITERS = PER_W // CHUNK

BETTA = N_BASIS ** 2 / R_MAX ** 2
RAD_NORM = (2.0 * BETTA / math.pi) ** 0.25
EMBED_NORM = 1.0 / math.sqrt(N_BASIS)
LN_A = math.log(RAD_NORM * EMBED_NORM)
SHIFTS = [R_MIN + (R_MAX - R_MIN) / N_BASIS * k for k in range(N_BASIS)]

CUT_K = (math.pi / 12.0) ** 2
CUT_POLY = (2.3153174155601487e-05, -0.001385366693302603,
            0.041663578930696436, -0.4999990506281048, 0.9999999532476085)


def _radial_sc_body(dr_hbm, zi_hbm, zj_hbm, tab_hbm, out_hbm,
                    dr_v, zi_v, zj_v, idx_v, rows_v, out_v, sem_in, sem_g):
    cid = lax.axis_index("c")
    sid = lax.axis_index("s")
    wid = sid * NC + cid
    base = wid * PER_W
    lane = lax.iota(jnp.int32, 16)

    def chunk(j, carry):
        cb = pl.multiple_of(base + j * CHUNK, 8)
        cp_dr = pltpu.async_copy(dr_hbm.at[pl.ds(cb, CHUNK)], dr_v, sem_in)
        cp_zi = pltpu.async_copy(zi_hbm.at[pl.ds(cb, CHUNK)], zi_v, sem_in)
        cp_zj = pltpu.async_copy(zj_hbm.at[pl.ds(cb, CHUNK)], zj_v, sem_in)
        cp_dr.wait()
        cp_zi.wait()
        cp_zj.wait()
        for g in range(NG):
            sl = pl.ds(g * 16, 16)
            idx_v[g // NSUB, pl.ds((g % NSUB) * 16, 16)] = (
                zj_v[sl] * N_SPECIES + zi_v[sl])
        gathers = []
        for s in range(NSUB):
            gathers.append(pltpu.async_copy(
                tab_hbm.at[idx_v.at[s]],
                rows_v.at[pl.ds(s * SUB, SUB)], sem_g))
        for cp in gathers:
            cp.wait()
        for g in range(NG):
            sl = pl.ds(g * 16, 16)
            drv = dr_v[sl]
            rows = lane + g * 16
            basis = []
            for k in range(N_BASIS):
                d = drv - SHIFTS[k]
                basis.append(jnp.exp((LN_A - BETTA * d * d).astype(jnp.float32)))
            t = drv * drv * CUT_K
            c = jnp.float32(CUT_POLY[0])
            for coef in CUT_POLY[1:]:
                c = c * t + jnp.float32(coef)
            cut = c * c
            for r in range(N_RADIAL):
                acc = plsc.load_gather(
                    rows_v, [rows, jnp.full((16,), r * N_BASIS, jnp.int32)]
                ) * basis[0]
                for k in range(1, N_BASIS):
                    acc = acc + plsc.load_gather(
                        rows_v,
                        [rows, jnp.full((16,), r * N_BASIS + k, jnp.int32)],
                    ) * basis[k]
                plsc.store_scatter(
                    out_v, [rows, jnp.full((16,), r, jnp.int32)], acc * cut)
        pltpu.sync_copy(out_v, out_hbm.at[pl.ds(cb, CHUNK)])
        return carry

    lax.fori_loop(0, ITERS, chunk, 0)


_radial_sc = pl.kernel(
    _radial_sc_body,
    out_type=jax.ShapeDtypeStruct((N_NBRS, N_RADIAL), jnp.float32),
    mesh=plsc.VectorSubcoreMesh(core_axis_name="c", subcore_axis_name="s",
                                num_cores=NC, num_subcores=NS),
    scratch_types=[
        pltpu.VMEM((CHUNK,), jnp.float32),
        pltpu.VMEM((CHUNK,), jnp.int32),
        pltpu.VMEM((CHUNK,), jnp.int32),
        pltpu.VMEM((NSUB, SUB), jnp.int32),
        pltpu.VMEM((CHUNK, ROW_PAD), jnp.float32),
        pltpu.VMEM((CHUNK, N_RADIAL), jnp.float32),
        pltpu.SemaphoreType.DMA,
        pltpu.SemaphoreType.DMA,
    ],
    compiler_params=pltpu.CompilerParams(use_tc_tiling_on_sc=False,
                                         needs_layout_passes=False),
)


def kernel(dr, Z_i, Z_j, embeddings):
    table = embeddings.reshape(N_SPECIES * N_SPECIES, N_RADIAL * N_BASIS)
    table = jnp.pad(table, ((0, 0), (0, ROW_PAD - N_RADIAL * N_BASIS)))
    return _radial_sc(dr.astype(jnp.float32), Z_i, Z_j, table)


# speedup vs baseline: 24.6896x; 1.3904x over previous
import math

import jax
import jax.numpy as jnp
from jax import lax
from jax.experimental import pallas as pl
from jax.experimental.pallas import tpu as pltpu
from jax.experimental.pallas import tpu_sc as plsc

N_NBRS = 1600000
N_SPECIES = 119
N_RADIAL = 5
N_BASIS = 7
R_MIN = 0.5
R_MAX = 6.0

ROW = N_RADIAL * N_BASIS
ROW_PAD = 48

NC = 2
NS = 16
NW = NC * NS
PER_W = N_NBRS // NW
SUB = 80
NSUB = 25
GPS = SUB // 16
CHUNK = SUB * NSUB
ITERS = PER_W // CHUNK

BETTA = N_BASIS ** 2 / R_MAX ** 2
RAD_NORM = (2.0 * BETTA / math.pi) ** 0.25
EMBED_NORM = 1.0 / math.sqrt(N_BASIS)
LN_A = math.log(RAD_NORM * EMBED_NORM)
SHIFTS = [R_MIN + (R_MAX - R_MIN) / N_BASIS * k for k in range(N_BASIS)]

CUT_K = (math.pi / 12.0) ** 2
CUT_POLY = (2.3153174155601487e-05, -0.001385366693302603,
            0.041663578930696436, -0.4999990506281048, 0.9999999532476085)


def _radial_sc_body(dr_hbm, zi_hbm, zj_hbm, tab_hbm, out_hbm,
                    dr_v, zi_v, zj_v, idx_v, rows_v, out_v, sem_in, sem_g):
    cid = lax.axis_index("c")
    sid = lax.axis_index("s")
    wid = sid * NC + cid
    base = wid * PER_W
    lane = lax.iota(jnp.int32, 16)

    def chunk(j, carry):
        cb = pl.multiple_of(base + j * CHUNK, 8)
        cp_dr = pltpu.async_copy(dr_hbm.at[pl.ds(cb, CHUNK)], dr_v, sem_in)
        cp_zi = pltpu.async_copy(zi_hbm.at[pl.ds(cb, CHUNK)], zi_v, sem_in)
        cp_zj = pltpu.async_copy(zj_hbm.at[pl.ds(cb, CHUNK)], zj_v, sem_in)
        cp_dr.wait()
        cp_zi.wait()
        cp_zj.wait()

        def fire(s, carry2):
            sb = s * SUB
            for gg in range(GPS):
                sl = pl.ds(sb + gg * 16, 16)
                idx_v[s, pl.ds(gg * 16, 16)] = (
                    zj_v[sl] * N_SPECIES + zi_v[sl])
            pltpu.async_copy(
                tab_hbm.at[idx_v.at[s]], rows_v.at[pl.ds(sb, SUB)], sem_g)
            return carry2

        lax.fori_loop(0, NSUB, fire, 0)

        def consume(s, carry2):
            sb = s * SUB
            pltpu.make_async_copy(
                tab_hbm.at[idx_v.at[s]], rows_v.at[pl.ds(sb, SUB)],
                sem_g).wait()
            for gg in range(GPS):
                sl = pl.ds(sb + gg * 16, 16)
                drv = dr_v[sl]
                rows = lane + (sb + gg * 16)
                basis = []
                for k in range(N_BASIS):
                    d = drv - SHIFTS[k]
                    basis.append(
                        jnp.exp((LN_A - BETTA * d * d).astype(jnp.float32)))
                t = drv * drv * CUT_K
                c = jnp.float32(CUT_POLY[0])
                for coef in CUT_POLY[1:]:
                    c = c * t + jnp.float32(coef)
                cut = c * c
                for r in range(N_RADIAL):
                    acc = plsc.load_gather(
                        rows_v, [rows, jnp.full((16,), r * N_BASIS, jnp.int32)]
                    ) * basis[0]
                    for k in range(1, N_BASIS):
                        acc = acc + plsc.load_gather(
                            rows_v,
                            [rows, jnp.full((16,), r * N_BASIS + k, jnp.int32)],
                        ) * basis[k]
                    plsc.store_scatter(
                        out_v, [rows, jnp.full((16,), r, jnp.int32)], acc * cut)
            return carry2

        lax.fori_loop(0, NSUB, consume, 0)
        pltpu.sync_copy(out_v, out_hbm.at[pl.ds(cb, CHUNK)])
        return carry

    lax.fori_loop(0, ITERS, chunk, 0)


_radial_sc = pl.kernel(
    _radial_sc_body,
    out_type=jax.ShapeDtypeStruct((N_NBRS, N_RADIAL), jnp.float32),
    mesh=plsc.VectorSubcoreMesh(core_axis_name="c", subcore_axis_name="s",
                                num_cores=NC, num_subcores=NS),
    scratch_types=[
        pltpu.VMEM((CHUNK,), jnp.float32),
        pltpu.VMEM((CHUNK,), jnp.int32),
        pltpu.VMEM((CHUNK,), jnp.int32),
        pltpu.VMEM((NSUB, SUB), jnp.int32),
        pltpu.VMEM((CHUNK, ROW_PAD), jnp.float32),
        pltpu.VMEM((CHUNK, N_RADIAL), jnp.float32),
        pltpu.SemaphoreType.DMA,
        pltpu.SemaphoreType.DMA,
    ],
    compiler_params=pltpu.CompilerParams(use_tc_tiling_on_sc=False,
                                         needs_layout_passes=False),
)


def kernel(dr, Z_i, Z_j, embeddings):
    table = embeddings.reshape(N_SPECIES * N_SPECIES, ROW)
    table = jnp.pad(table, ((0, 0), (0, ROW_PAD - ROW)))
    return _radial_sc(dr.astype(jnp.float32), Z_i, Z_j, table)
